# baseline (speedup 1.0000x reference)
import functools

import jax
import jax.numpy as jnp
from jax import lax
from jax.experimental import pallas as pl
from jax.experimental.pallas import tpu as pltpu
from jax.experimental.pallas import tpu_sc as plsc

N = 262144
MAX_RES = 512
MIN_RES = 16
MIN_R = 0.08
MAX_R = 0.5
L = 6

_DIFF_OFF = 2096640
_TTOT = 2098176
_T3 = 3 * _TTOT
_SIZES3 = [6 * (512 >> l) * (512 >> l) * 3 for l in range(L)] + [4608]
_OFFS3 = [3 * (2097152 - (2097152 >> (2 * l))) for l in range(L)] + [3 * _DIFF_OFF]

NW = 32
RAYS_PER_W = N // NW
C = 512
NG = C // 16
NCHUNK = RAYS_PER_W // C
ISLOTS = 12 * C
GSLOTS = 36 * C
DMA_B = 512
NDMA = ISLOTS // DMA_B


def _face_uv(x, y, z):
    ax, ay, az = jnp.abs(x), jnp.abs(y), jnp.abs(z)
    cx = (ax >= ay) & (ax >= az)
    cy = ((ax < ay) | (ax < az)) & (ay >= az)
    ma = jnp.maximum(jnp.where(cx, ax, jnp.where(cy, ay, az)), 1e-12)
    face = jnp.where(cx, jnp.where(x > 0, 0, 1),
                     jnp.where(cy, jnp.where(y > 0, 2, 3),
                               jnp.where(z > 0, 4, 5))).astype(jnp.int32)
    sc = jnp.where(cx, jnp.where(x > 0, -z, z),
                   jnp.where(cy, x, jnp.where(z > 0, x, -x)))
    tc = jnp.where(cx, -y, jnp.where(cy, jnp.where(y > 0, z, -z), -y))
    u = 0.5 * (sc / ma + 1.0)
    v = 0.5 * (tc / ma + 1.0)
    return face, u, v


def _bilerp_diffuse_idx(face, u, v):
    R = 16
    fu = u * float(R) - 0.5
    fv = v * float(R) - 0.5
    u0 = jnp.clip(fu.astype(jnp.int32), 0, R - 1)
    v0 = jnp.clip(fv.astype(jnp.int32), 0, R - 1)
    du = jnp.minimum(u0 + 1, R - 1) - u0
    dv = jnp.minimum(v0 + 1, R - 1) - v0
    wu = jnp.clip(fu - u0.astype(jnp.float32), 0.0, 1.0)
    wv = jnp.clip(fv - v0.astype(jnp.float32), 0.0, 1.0)
    i00 = _DIFF_OFF + face * (R * R) + v0 * R + u0
    i01 = i00 + du
    i10 = i00 + dv * R
    i11 = i10 + du
    return i00, i01, i10, i11, wu, wv


def _bilerp_mip_idx(face, u, v, lvl):
    sh1 = 9 - lvl
    sh2 = 18 - 2 * lvl
    Ri = jnp.left_shift(1, sh1)
    Rm1 = Ri - 1
    Rf = Ri.astype(jnp.float32)
    fu = u * Rf - 0.5
    fv = v * Rf - 0.5
    u0 = jnp.clip(fu.astype(jnp.int32), 0, Rm1)
    v0 = jnp.clip(fv.astype(jnp.int32), 0, Rm1)
    du = jnp.minimum(u0 + 1, Rm1) - u0
    dv = jnp.minimum(v0 + 1, Rm1) - v0
    wu = jnp.clip(fu - u0.astype(jnp.float32), 0.0, 1.0)
    wv = jnp.clip(fv - v0.astype(jnp.float32), 0.0, 1.0)
    off = 2097152 - jnp.right_shift(2097152, 2 * lvl)
    i00 = off + jnp.left_shift(face, sh2) + jnp.left_shift(v0, sh1) + u0
    i01 = i00 + du
    i10 = i00 + jnp.left_shift(dv, sh1)
    i11 = i10 + du
    return i00, i01, i10, i11, wu, wv


def _get_mip(r):
    return jnp.where(
        r < MAX_R,
        (jnp.clip(r, MIN_R, MAX_R) - MIN_R) / (MAX_R - MIN_R) * (L - 2),
        (jnp.clip(r, MAX_R, 1.0) - MAX_R) / (1.0 - MAX_R) + (L - 2))


def _sc_body(rays_hbm, tab_r, tab_g, tab_b, out_hbm,
             rays_v, idx_v, rows_v, w_v, out_v, gsem):
    core = lax.axis_index("c")
    sub = lax.axis_index("s")
    wid = sub * 2 + core
    base0 = wid * RAYS_PER_W

    def chunk_body(ci, carry):
        base = base0 + ci * C
        pltpu.sync_copy(rays_hbm.at[:, pl.ds(base, C)], rays_v)

        def stage1(g, carry):
            s = g * 16

            def put(t, texel_idx):
                idx_v[pl.ds(t * C + s, 16)] = texel_idx

            nx = rays_v[0, pl.ds(s, 16)]
            ny = rays_v[1, pl.ds(s, 16)]
            nz = rays_v[2, pl.ds(s, 16)]
            fd, ud, vd = _face_uv(nx, ny, nz)
            d00, d01, d10, d11, wud, wvd = _bilerp_diffuse_idx(fd, ud, vd)
            put(0, d00)
            put(1, d01)
            put(2, d10)
            put(3, d11)
            w_v[0, pl.ds(s, 16)] = wud
            w_v[1, pl.ds(s, 16)] = wvd

            rx = rays_v[3, pl.ds(s, 16)]
            ry = rays_v[4, pl.ds(s, 16)]
            rz = rays_v[5, pl.ds(s, 16)]
            rough = rays_v[6, pl.ds(s, 16)]
            fs, us, vs = _face_uv(rx, ry, rz)
            mip = _get_mip(rough)
            l0 = jnp.minimum(mip.astype(jnp.int32), L - 1)
            frac = mip - l0.astype(jnp.float32)
            l1 = jnp.minimum(l0 + 1, L - 1)
            a00, a01, a10, a11, wu0, wv0 = _bilerp_mip_idx(fs, us, vs, l0)
            put(4, a00)
            put(5, a01)
            put(6, a10)
            put(7, a11)
            b00, b01, b10, b11, wu1, wv1 = _bilerp_mip_idx(fs, us, vs, l1)
            put(8, b00)
            put(9, b01)
            put(10, b10)
            put(11, b11)
            w_v[2, pl.ds(s, 16)] = wu0
            w_v[3, pl.ds(s, 16)] = wv0
            w_v[4, pl.ds(s, 16)] = wu1
            w_v[5, pl.ds(s, 16)] = wv1
            w_v[6, pl.ds(s, 16)] = frac
            return carry

        lax.fori_loop(0, NG, stage1, 0)

        copies = [
            pltpu.async_copy(
                tab.at[idx_v.at[pl.ds(j * DMA_B, DMA_B)]],
                rows_v.at[pl.ds(ch * ISLOTS + j * DMA_B, DMA_B)], gsem)
            for ch, tab in enumerate((tab_r, tab_g, tab_b))
            for j in range(NDMA)
        ]
        for cp in copies:
            cp.wait()

        def stage2(g, carry):
            s = g * 16
            wud = w_v[0, pl.ds(s, 16)]
            wvd = w_v[1, pl.ds(s, 16)]
            wu0 = w_v[2, pl.ds(s, 16)]
            wv0 = w_v[3, pl.ds(s, 16)]
            wu1 = w_v[4, pl.ds(s, 16)]
            wv1 = w_v[5, pl.ds(s, 16)]
            frac = w_v[6, pl.ds(s, 16)]
            for c in range(3):

                def tap(t):
                    return rows_v[pl.ds(c * ISLOTS + t * C + s, 16)]

                q00, q01, q10, q11 = tap(0), tap(1), tap(2), tap(3)
                top = q00 * (1.0 - wud) + q01 * wud
                bot = q10 * (1.0 - wud) + q11 * wud
                out_v[c, pl.ds(s, 16)] = top * (1.0 - wvd) + bot * wvd

                q00, q01, q10, q11 = tap(4), tap(5), tap(6), tap(7)
                top = q00 * (1.0 - wu0) + q01 * wu0
                bot = q10 * (1.0 - wu0) + q11 * wu0
                s0 = top * (1.0 - wv0) + bot * wv0
                q00, q01, q10, q11 = tap(8), tap(9), tap(10), tap(11)
                top = q00 * (1.0 - wu1) + q01 * wu1
                bot = q10 * (1.0 - wu1) + q11 * wu1
                s1 = top * (1.0 - wv1) + bot * wv1
                out_v[3 + c, pl.ds(s, 16)] = s0 * (1.0 - frac) + s1 * frac
            return carry

        lax.fori_loop(0, NG, stage2, 0)
        pltpu.sync_copy(out_v, out_hbm.at[:, pl.ds(base, C)])
        return carry

    lax.fori_loop(0, NCHUNK, chunk_body, 0)


@functools.cache
def _sc_lookup():
    return pl.kernel(
        _sc_body,
        mesh=plsc.VectorSubcoreMesh(core_axis_name="c", subcore_axis_name="s"),
        out_type=jax.ShapeDtypeStruct((6, N), jnp.float32),
        scratch_types=[
            pltpu.VMEM((7, C), jnp.float32),
            pltpu.VMEM((ISLOTS,), jnp.int32),
            pltpu.VMEM((GSLOTS,), jnp.float32),
            pltpu.VMEM((7, C), jnp.float32),
            pltpu.VMEM((6, C), jnp.float32),
            pltpu.SemaphoreType.DMA,
        ],
    )


def _face_dirs(R):
    g = (jnp.arange(R, dtype=jnp.float32) + 0.5) / R * 2.0 - 1.0
    tc, sc = jnp.meshgrid(g, g, indexing='ij')
    one = jnp.ones_like(sc)
    d0 = jnp.stack([one, -tc, -sc], -1)
    d1 = jnp.stack([-one, -tc, sc], -1)
    d2 = jnp.stack([sc, one, tc], -1)
    d3 = jnp.stack([sc, -one, -tc], -1)
    d4 = jnp.stack([sc, -tc, one], -1)
    d5 = jnp.stack([-sc, -tc, -one], -1)
    dirs = jnp.stack([d0, d1, d2, d3, d4, d5], 0)
    w = 1.0 / (sc ** 2 + tc ** 2 + 1.0) ** 1.5
    w6 = jnp.broadcast_to(w, (6, R, R))
    n = dirs / jnp.linalg.norm(dirs, axis=-1, keepdims=True)
    return n.reshape(-1, 3), w6.reshape(-1)


def _diffuse_cubemap(tex):
    R = tex.shape[1]
    dirs, w = _face_dirs(R)
    Cm = tex.reshape(-1, 3)
    cos = jnp.maximum(dirs @ dirs.T, 0.0)
    cw = cos * w[None, :]
    return ((cw @ Cm) / jnp.maximum(cw.sum(-1, keepdims=True), 1e-8)
            ).reshape(6, R, R, 3)


def _build_mips(base):
    mips = [base]
    while mips[-1].shape[1] > MIN_RES:
        m = mips[-1]
        R = m.shape[1]
        mips.append(m.reshape(6, R // 2, 2, R // 2, 2, 3).mean(axis=(2, 4)))
    diffuse = _diffuse_cubemap(mips[-1])
    return mips, diffuse


def kernel(shading_normal, reflective, roughness, base):
    mips, diffuse = _build_mips(base)
    rows3 = jnp.concatenate(
        [m.reshape(-1, 3) for m in mips] + [diffuse.reshape(-1, 3)], 0)
    planes = rows3.T
    rays = jnp.concatenate(
        [shading_normal.T, reflective.T, roughness.T], 0)
    out6 = _sc_lookup()(rays, planes[0], planes[1], planes[2])
    return out6[0:3].T, out6[3:6].T


# speedup vs baseline: 1.0344x; 1.0344x over previous
import functools

import jax
import jax.numpy as jnp
from jax import lax
from jax.experimental import pallas as pl
from jax.experimental.pallas import tpu as pltpu
from jax.experimental.pallas import tpu_sc as plsc

N = 262144
MAX_RES = 512
MIN_RES = 16
MIN_R = 0.08
MAX_R = 0.5
L = 6

_DIFF_OFF = 2096640
_TTOT = 2098176
_T3 = 3 * _TTOT
_SIZES3 = [6 * (512 >> l) * (512 >> l) * 3 for l in range(L)] + [4608]
_OFFS3 = [3 * (2097152 - (2097152 >> (2 * l))) for l in range(L)] + [3 * _DIFF_OFF]

NW = 32
RAYS_PER_W = N // NW
C = 256
NG = C // 16
NCHUNK = RAYS_PER_W // C
ISLOTS = 12 * C
GSLOTS = 36 * C
DMA_B = 512
NDMA = ISLOTS // DMA_B


def _face_uv(x, y, z):
    ax, ay, az = jnp.abs(x), jnp.abs(y), jnp.abs(z)
    cx = (ax >= ay) & (ax >= az)
    cy = ((ax < ay) | (ax < az)) & (ay >= az)
    ma = jnp.maximum(jnp.where(cx, ax, jnp.where(cy, ay, az)), 1e-12)
    face = jnp.where(cx, jnp.where(x > 0, 0, 1),
                     jnp.where(cy, jnp.where(y > 0, 2, 3),
                               jnp.where(z > 0, 4, 5))).astype(jnp.int32)
    sc = jnp.where(cx, jnp.where(x > 0, -z, z),
                   jnp.where(cy, x, jnp.where(z > 0, x, -x)))
    tc = jnp.where(cx, -y, jnp.where(cy, jnp.where(y > 0, z, -z), -y))
    u = 0.5 * (sc / ma + 1.0)
    v = 0.5 * (tc / ma + 1.0)
    return face, u, v


def _bilerp_diffuse_idx(face, u, v):
    R = 16
    fu = u * float(R) - 0.5
    fv = v * float(R) - 0.5
    u0 = jnp.clip(fu.astype(jnp.int32), 0, R - 1)
    v0 = jnp.clip(fv.astype(jnp.int32), 0, R - 1)
    du = jnp.minimum(u0 + 1, R - 1) - u0
    dv = jnp.minimum(v0 + 1, R - 1) - v0
    wu = jnp.clip(fu - u0.astype(jnp.float32), 0.0, 1.0)
    wv = jnp.clip(fv - v0.astype(jnp.float32), 0.0, 1.0)
    i00 = _DIFF_OFF + face * (R * R) + v0 * R + u0
    i01 = i00 + du
    i10 = i00 + dv * R
    i11 = i10 + du
    return i00, i01, i10, i11, wu, wv


def _bilerp_mip_idx(face, u, v, lvl):
    sh1 = 9 - lvl
    sh2 = 18 - 2 * lvl
    Ri = jnp.left_shift(1, sh1)
    Rm1 = Ri - 1
    Rf = Ri.astype(jnp.float32)
    fu = u * Rf - 0.5
    fv = v * Rf - 0.5
    u0 = jnp.clip(fu.astype(jnp.int32), 0, Rm1)
    v0 = jnp.clip(fv.astype(jnp.int32), 0, Rm1)
    du = jnp.minimum(u0 + 1, Rm1) - u0
    dv = jnp.minimum(v0 + 1, Rm1) - v0
    wu = jnp.clip(fu - u0.astype(jnp.float32), 0.0, 1.0)
    wv = jnp.clip(fv - v0.astype(jnp.float32), 0.0, 1.0)
    off = 2097152 - jnp.right_shift(2097152, 2 * lvl)
    i00 = off + jnp.left_shift(face, sh2) + jnp.left_shift(v0, sh1) + u0
    i01 = i00 + du
    i10 = i00 + jnp.left_shift(dv, sh1)
    i11 = i10 + du
    return i00, i01, i10, i11, wu, wv


def _get_mip(r):
    return jnp.where(
        r < MAX_R,
        (jnp.clip(r, MIN_R, MAX_R) - MIN_R) / (MAX_R - MIN_R) * (L - 2),
        (jnp.clip(r, MAX_R, 1.0) - MAX_R) / (1.0 - MAX_R) + (L - 2))


def _sc_body(rays_hbm, tab_r, tab_g, tab_b, out_hbm,
             rays_v0, rays_v1, idx_v0, idx_v1, rows_v0, rows_v1,
             w_v0, w_v1, out_v0, out_v1, gsem0, gsem1):
    core = lax.axis_index("c")
    sub = lax.axis_index("s")
    wid = sub * 2 + core
    base0 = wid * RAYS_PER_W
    rays_vs = (rays_v0, rays_v1)
    idx_vs = (idx_v0, idx_v1)
    rows_vs = (rows_v0, rows_v1)
    w_vs = (w_v0, w_v1)
    out_vs = (out_v0, out_v1)
    gsems = (gsem0, gsem1)

    def run_stage1(b):
        rays_v, idx_v, w_v = rays_vs[b], idx_vs[b], w_vs[b]

        def stage1(g, carry):
            s = g * 16

            def put(t, texel_idx):
                idx_v[pl.ds(t * C + s, 16)] = texel_idx

            nx = rays_v[0, pl.ds(s, 16)]
            ny = rays_v[1, pl.ds(s, 16)]
            nz = rays_v[2, pl.ds(s, 16)]
            fd, ud, vd = _face_uv(nx, ny, nz)
            d00, d01, d10, d11, wud, wvd = _bilerp_diffuse_idx(fd, ud, vd)
            put(0, d00)
            put(1, d01)
            put(2, d10)
            put(3, d11)
            w_v[0, pl.ds(s, 16)] = wud
            w_v[1, pl.ds(s, 16)] = wvd

            rx = rays_v[3, pl.ds(s, 16)]
            ry = rays_v[4, pl.ds(s, 16)]
            rz = rays_v[5, pl.ds(s, 16)]
            rough = rays_v[6, pl.ds(s, 16)]
            fs, us, vs = _face_uv(rx, ry, rz)
            mip = _get_mip(rough)
            l0 = jnp.minimum(mip.astype(jnp.int32), L - 1)
            frac = mip - l0.astype(jnp.float32)
            l1 = jnp.minimum(l0 + 1, L - 1)
            a00, a01, a10, a11, wu0, wv0 = _bilerp_mip_idx(fs, us, vs, l0)
            put(4, a00)
            put(5, a01)
            put(6, a10)
            put(7, a11)
            b00, b01, b10, b11, wu1, wv1 = _bilerp_mip_idx(fs, us, vs, l1)
            put(8, b00)
            put(9, b01)
            put(10, b10)
            put(11, b11)
            w_v[2, pl.ds(s, 16)] = wu0
            w_v[3, pl.ds(s, 16)] = wv0
            w_v[4, pl.ds(s, 16)] = wu1
            w_v[5, pl.ds(s, 16)] = wv1
            w_v[6, pl.ds(s, 16)] = frac
            return carry

        lax.fori_loop(0, NG, stage1, 0)

    def issue_gathers(b):
        idx_v, rows_v, gsem = idx_vs[b], rows_vs[b], gsems[b]
        return [
            pltpu.async_copy(
                tab.at[idx_v.at[pl.ds(j * DMA_B, DMA_B)]],
                rows_v.at[pl.ds(ch * ISLOTS + j * DMA_B, DMA_B)], gsem)
            for ch, tab in enumerate((tab_r, tab_g, tab_b))
            for j in range(NDMA)
        ]

    def run_stage2(b):
        rows_v, w_v, out_v = rows_vs[b], w_vs[b], out_vs[b]

        def stage2(g, carry):
            s = g * 16
            wud = w_v[0, pl.ds(s, 16)]
            wvd = w_v[1, pl.ds(s, 16)]
            wu0 = w_v[2, pl.ds(s, 16)]
            wv0 = w_v[3, pl.ds(s, 16)]
            wu1 = w_v[4, pl.ds(s, 16)]
            wv1 = w_v[5, pl.ds(s, 16)]
            frac = w_v[6, pl.ds(s, 16)]
            for c in range(3):

                def tap(t):
                    return rows_v[pl.ds(c * ISLOTS + t * C + s, 16)]

                q00, q01, q10, q11 = tap(0), tap(1), tap(2), tap(3)
                top = q00 * (1.0 - wud) + q01 * wud
                bot = q10 * (1.0 - wud) + q11 * wud
                out_v[c, pl.ds(s, 16)] = top * (1.0 - wvd) + bot * wvd

                q00, q01, q10, q11 = tap(4), tap(5), tap(6), tap(7)
                top = q00 * (1.0 - wu0) + q01 * wu0
                bot = q10 * (1.0 - wu0) + q11 * wu0
                s0 = top * (1.0 - wv0) + bot * wv0
                q00, q01, q10, q11 = tap(8), tap(9), tap(10), tap(11)
                top = q00 * (1.0 - wu1) + q01 * wu1
                bot = q10 * (1.0 - wu1) + q11 * wu1
                s1 = top * (1.0 - wv1) + bot * wv1
                out_v[3 + c, pl.ds(s, 16)] = s0 * (1.0 - frac) + s1 * frac
            return carry

        lax.fori_loop(0, NG, stage2, 0)

    def pair_body(ci, carry):
        i0 = 2 * ci
        pltpu.sync_copy(rays_hbm.at[:, pl.ds(base0 + i0 * C, C)],
                        rays_vs[0])
        run_stage1(0)
        cps0 = issue_gathers(0)
        pltpu.sync_copy(rays_hbm.at[:, pl.ds(base0 + (i0 + 1) * C, C)],
                        rays_vs[1])
        run_stage1(1)
        cps1 = issue_gathers(1)
        for cp in cps0:
            cp.wait()
        run_stage2(0)
        pltpu.sync_copy(out_vs[0], out_hbm.at[:, pl.ds(base0 + i0 * C, C)])
        for cp in cps1:
            cp.wait()
        run_stage2(1)
        pltpu.sync_copy(out_vs[1],
                        out_hbm.at[:, pl.ds(base0 + (i0 + 1) * C, C)])
        return carry

    lax.fori_loop(0, NCHUNK // 2, pair_body, 0)


@functools.cache
def _sc_lookup():
    return pl.kernel(
        _sc_body,
        mesh=plsc.VectorSubcoreMesh(core_axis_name="c", subcore_axis_name="s"),
        out_type=jax.ShapeDtypeStruct((6, N), jnp.float32),
        scratch_types=[
            pltpu.VMEM((7, C), jnp.float32),
            pltpu.VMEM((7, C), jnp.float32),
            pltpu.VMEM((ISLOTS,), jnp.int32),
            pltpu.VMEM((ISLOTS,), jnp.int32),
            pltpu.VMEM((GSLOTS,), jnp.float32),
            pltpu.VMEM((GSLOTS,), jnp.float32),
            pltpu.VMEM((7, C), jnp.float32),
            pltpu.VMEM((7, C), jnp.float32),
            pltpu.VMEM((6, C), jnp.float32),
            pltpu.VMEM((6, C), jnp.float32),
            pltpu.SemaphoreType.DMA,
            pltpu.SemaphoreType.DMA,
        ],
    )


def _face_dirs(R):
    g = (jnp.arange(R, dtype=jnp.float32) + 0.5) / R * 2.0 - 1.0
    tc, sc = jnp.meshgrid(g, g, indexing='ij')
    one = jnp.ones_like(sc)
    d0 = jnp.stack([one, -tc, -sc], -1)
    d1 = jnp.stack([-one, -tc, sc], -1)
    d2 = jnp.stack([sc, one, tc], -1)
    d3 = jnp.stack([sc, -one, -tc], -1)
    d4 = jnp.stack([sc, -tc, one], -1)
    d5 = jnp.stack([-sc, -tc, -one], -1)
    dirs = jnp.stack([d0, d1, d2, d3, d4, d5], 0)
    w = 1.0 / (sc ** 2 + tc ** 2 + 1.0) ** 1.5
    w6 = jnp.broadcast_to(w, (6, R, R))
    n = dirs / jnp.linalg.norm(dirs, axis=-1, keepdims=True)
    return n.reshape(-1, 3), w6.reshape(-1)


def _diffuse_cubemap(tex):
    R = tex.shape[1]
    dirs, w = _face_dirs(R)
    Cm = tex.reshape(-1, 3)
    cos = jnp.maximum(dirs @ dirs.T, 0.0)
    cw = cos * w[None, :]
    return ((cw @ Cm) / jnp.maximum(cw.sum(-1, keepdims=True), 1e-8)
            ).reshape(6, R, R, 3)


def _build_mips(base):
    mips = [base]
    while mips[-1].shape[1] > MIN_RES:
        m = mips[-1]
        R = m.shape[1]
        mips.append(m.reshape(6, R // 2, 2, R // 2, 2, 3).mean(axis=(2, 4)))
    diffuse = _diffuse_cubemap(mips[-1])
    return mips, diffuse


def kernel(shading_normal, reflective, roughness, base):
    mips, diffuse = _build_mips(base)
    rows3 = jnp.concatenate(
        [m.reshape(-1, 3) for m in mips] + [diffuse.reshape(-1, 3)], 0)
    planes = rows3.T
    rays = jnp.concatenate(
        [shading_normal.T, reflective.T, roughness.T], 0)
    out6 = _sc_lookup()(rays, planes[0], planes[1], planes[2])
    return out6[0:3].T, out6[3:6].T
